# baseline (speedup 1.0000x reference)
import functools

import jax
import jax.numpy as jnp
from jax import lax
from jax.experimental import pallas as pl
from jax.experimental.pallas import tpu as pltpu
from jax.experimental.pallas import tpu_sc as plsc

_BLK = 128
_LANES = 16


def _pos_encoding(length: int, hidden: int) -> jax.Array:
    depth = hidden // 2
    positions = jnp.arange(length)[:, None].astype(jnp.float32)
    depths = jnp.arange(depth)[None, :].astype(jnp.float32) / depth
    angle_rates = 1.0 / (10000.0 ** depths)
    angle_rads = positions * angle_rates
    return jnp.concatenate(
        [jnp.sin(angle_rads), jnp.cos(angle_rads)], axis=-1
    ).astype(jnp.float32)


@functools.partial(jax.jit, static_argnames=("b_total", "hidden", "length"))
def _emb_lookup(xt2d, pos_bc, table2, *, b_total, hidden, length):
    info = plsc.get_sparse_core_info()
    nc, ns = info.num_cores, info.num_subcores
    nw = nc * ns
    n_btiles = b_total // _BLK
    n_chunks_total = length * n_btiles
    per_w = n_chunks_total // nw
    scale = float(hidden) ** 0.5
    hgrp = hidden // 8
    nbg = _BLK // _LANES

    mesh = plsc.VectorSubcoreMesh(core_axis_name="c", subcore_axis_name="s")

    @functools.partial(
        pl.kernel,
        mesh=mesh,
        compiler_params=pltpu.CompilerParams(
            use_tc_tiling_on_sc=False, needs_layout_passes=False),
        out_type=jax.ShapeDtypeStruct(
            (length, hgrp, n_btiles, 8, _BLK), jnp.float32),
        scratch_types=[
            pltpu.VMEM((per_w, _BLK), jnp.int32),
            pltpu.VMEM((2, _BLK), jnp.int32),
            pltpu.VMEM((_BLK, _BLK), jnp.float32),
            pltpu.VMEM((_BLK, _BLK), jnp.float32),
            pltpu.VMEM((hidden, _LANES), jnp.float32),
            pltpu.VMEM((hidden, _LANES), jnp.float32),
            pltpu.VMEM((hgrp, 8, _BLK), jnp.float32),
            pltpu.VMEM((hgrp, 8, _BLK), jnp.float32),
            pltpu.SemaphoreType.DMA,
            pltpu.SemaphoreType.DMA,
            pltpu.SemaphoreType.DMA,
            pltpu.SemaphoreType.DMA,
            pltpu.SemaphoreType.DMA,
            pltpu.SemaphoreType.DMA,
        ],
    )
    def k(x_hbm, pos_hbm, tab_hbm, out_hbm, idx_all, idx_g, rows0, rows1,
          posv0, posv1, ob0, ob1, sg0, sg1, sp0, sp1, so0, so1):
        wid = lax.axis_index("s") * nc + lax.axis_index("c")
        base_c = wid * per_w
        rows = (rows0, rows1)
        posv = (posv0, posv1)
        obuf = (ob0, ob1)
        sem_g = (sg0, sg1)
        sem_p = (sp0, sp1)
        sem_o = (so0, so1)

        pltpu.sync_copy(x_hbm.at[pl.ds(base_c, per_w)], idx_all)

        iota = lax.iota(jnp.int32, _LANES)
        row_vecs = [iota + bg * _LANES for bg in range(nbg)]

        def shift_idx(j, b):
            for bg in range(nbg):
                sl = pl.ds(bg * _LANES, _LANES)
                idx_g[b, sl] = lax.shift_right_logical(idx_all[j, sl], 1)

        def gather_copy(b):
            return pltpu.make_async_copy(
                tab_hbm.at[idx_g.at[b]], rows[b], sem_g[b])

        def pos_copy(j, b):
            l = lax.shift_right_logical(base_c + j, 5)
            return pltpu.make_async_copy(pos_hbm.at[l], posv[b], sem_p[b])

        def store_copy(j, b, g1):
            c = base_c + j
            l = lax.shift_right_logical(c, 5)
            g0b = lax.rem(c, n_btiles)
            return pltpu.make_async_copy(
                obuf[b].at[g1], out_hbm.at[l, g1, g0b], sem_o[b])

        def step(j, b):
            @pl.when(j + 1 < per_w)
            def _():
                shift_idx(j + 1, 1 - b)
                gather_copy(1 - b).start()
                pos_copy(j + 1, 1 - b).start()

            gather_copy(b).wait()
            pos_copy(j, b).wait()

            @pl.when(j >= 2)
            def _():
                for g1 in range(hgrp):
                    store_copy(j - 2, b, g1).wait()

            cols = [
                lax.shift_left(
                    lax.bitwise_and(
                        idx_all[j, pl.ds(bg * _LANES, _LANES)], 1), 6)
                for bg in range(nbg)
            ]
            rv = rows[b]
            ov = obuf[b]
            pv = posv[b]

            def hbody(h, carry):
                padd = pv[h]
                g1 = lax.shift_right_logical(h, 3)
                s = lax.bitwise_and(h, 7)
                for bg in range(nbg):
                    col = cols[bg] + h
                    src = plsc.load_gather(rv, [row_vecs[bg], col])
                    ov[g1, s, pl.ds(bg * _LANES, _LANES)] = (
                        src * scale + padd)
                return carry

            lax.fori_loop(0, hidden, hbody, 0, unroll=2)

            for g1 in range(hgrp):
                store_copy(j, b, g1).start()

        shift_idx(0, 0)
        gather_copy(0).start()
        pos_copy(0, 0).start()

        def pair(i, carry):
            step(2 * i + 0, 0)
            step(2 * i + 1, 1)
            return carry

        lax.fori_loop(0, per_w // 2, pair, 0)
        for g1 in range(hgrp):
            store_copy(per_w - 2, 0, g1).wait()
        for g1 in range(hgrp):
            store_copy(per_w - 1, 1, g1).wait()

    return k(xt2d, pos_bc, table2)


def kernel(x, table):
    b_total, length = x.shape
    hidden = table.shape[1]
    pos = _pos_encoding(length, hidden)
    pos_bc = jnp.broadcast_to(
        pos[:, :, None], (length, hidden, _LANES))
    xt2d = x.T.reshape(length * b_total // _BLK, _BLK)
    table2 = table.reshape(table.shape[0] // 2, 2 * hidden)
    out5 = _emb_lookup(
        xt2d, pos_bc, table2,
        b_total=b_total, hidden=hidden, length=length,
    )
    return (out5.transpose(2, 4, 0, 1, 3)
            .reshape(b_total, length, hidden))


# speedup vs baseline: 1.1213x; 1.1213x over previous
import functools

import jax
import jax.numpy as jnp
from jax import lax
from jax.experimental import pallas as pl
from jax.experimental.pallas import tpu as pltpu
from jax.experimental.pallas import tpu_sc as plsc

_BLK = 128
_LANES = 16
_PITCH = 72


def _pos_encoding(length: int, hidden: int) -> jax.Array:
    depth = hidden // 2
    positions = jnp.arange(length)[:, None].astype(jnp.float32)
    depths = jnp.arange(depth)[None, :].astype(jnp.float32) / depth
    angle_rates = 1.0 / (10000.0 ** depths)
    angle_rads = positions * angle_rates
    return jnp.concatenate(
        [jnp.sin(angle_rads), jnp.cos(angle_rads)], axis=-1
    ).astype(jnp.float32)


@functools.partial(jax.jit, static_argnames=("b_total", "hidden", "length"))
def _emb_lookup(xt2d, pos_bc, table72, *, b_total, hidden, length):
    info = plsc.get_sparse_core_info()
    nc, ns = info.num_cores, info.num_subcores
    nw = nc * ns
    n_btiles = b_total // _BLK
    n_chunks_total = length * n_btiles
    per_w = n_chunks_total // nw
    scale = float(hidden) ** 0.5
    hgrp = hidden // 8
    nbg = _BLK // _LANES

    mesh = plsc.VectorSubcoreMesh(core_axis_name="c", subcore_axis_name="s")

    @functools.partial(
        pl.kernel,
        mesh=mesh,
        compiler_params=pltpu.CompilerParams(
            use_tc_tiling_on_sc=False, needs_layout_passes=False),
        out_type=jax.ShapeDtypeStruct(
            (length, hgrp, n_btiles, 8, _BLK), jnp.float32),
        scratch_types=[
            pltpu.VMEM((per_w, _BLK), jnp.int32),
            pltpu.VMEM((_BLK, _PITCH), jnp.float32),
            pltpu.VMEM((_BLK, _PITCH), jnp.float32),
            pltpu.VMEM((hidden, _LANES), jnp.float32),
            pltpu.VMEM((hidden, _LANES), jnp.float32),
            pltpu.VMEM((hgrp, 8, _BLK), jnp.float32),
            pltpu.VMEM((hgrp, 8, _BLK), jnp.float32),
            pltpu.SemaphoreType.DMA,
            pltpu.SemaphoreType.DMA,
            pltpu.SemaphoreType.DMA,
            pltpu.SemaphoreType.DMA,
            pltpu.SemaphoreType.DMA,
            pltpu.SemaphoreType.DMA,
        ],
    )
    def k(x_hbm, pos_hbm, tab_hbm, out_hbm, idx_all, rows0, rows1,
          posv0, posv1, ob0, ob1, sg0, sg1, sp0, sp1, so0, so1):
        wid = lax.axis_index("s") * nc + lax.axis_index("c")
        base_c = wid * per_w
        rows = (rows0, rows1)
        posv = (posv0, posv1)
        obuf = (ob0, ob1)
        sem_g = (sg0, sg1)
        sem_p = (sp0, sp1)
        sem_o = (so0, so1)

        pltpu.sync_copy(x_hbm.at[pl.ds(base_c, per_w)], idx_all)

        iota = lax.iota(jnp.int32, _LANES)
        row_vecs = [iota + bg * _LANES for bg in range(nbg)]

        def gather_copy(j, b):
            return pltpu.make_async_copy(
                tab_hbm.at[idx_all.at[j]], rows[b], sem_g[b])

        def pos_copy(j, b):
            l = lax.shift_right_logical(base_c + j, 5)
            return pltpu.make_async_copy(pos_hbm.at[l], posv[b], sem_p[b])

        def store_copy(j, b, g1):
            c = base_c + j
            l = lax.shift_right_logical(c, 5)
            g0b = lax.rem(c, n_btiles)
            return pltpu.make_async_copy(
                obuf[b].at[g1], out_hbm.at[l, g1, g0b], sem_o[b])

        def step(j, b):
            @pl.when(j + 1 < per_w)
            def _():
                gather_copy(j + 1, 1 - b).start()
                pos_copy(j + 1, 1 - b).start()

            gather_copy(j, b).wait()
            pos_copy(j, b).wait()

            @pl.when(j >= 2)
            def _():
                for g1 in range(hgrp):
                    store_copy(j - 2, b, g1).wait()

            rv = rows[b]
            ov = obuf[b]
            pv = posv[b]

            def hbody(h, carry):
                padd = pv[h]
                hsplat = jnp.zeros((_LANES,), jnp.int32) + h
                g1 = lax.shift_right_logical(h, 3)
                s = lax.bitwise_and(h, 7)
                for bg in range(nbg):
                    src = plsc.load_gather(rv, [row_vecs[bg], hsplat])
                    ov[g1, s, pl.ds(bg * _LANES, _LANES)] = (
                        src * scale + padd)
                return carry

            lax.fori_loop(0, hidden, hbody, 0, unroll=2)

            for g1 in range(hgrp):
                store_copy(j, b, g1).start()

        gather_copy(0, 0).start()
        pos_copy(0, 0).start()

        def pair(i, carry):
            step(2 * i + 0, 0)
            step(2 * i + 1, 1)
            return carry

        lax.fori_loop(0, per_w // 2, pair, 0)
        for g1 in range(hgrp):
            store_copy(per_w - 2, 0, g1).wait()
        for g1 in range(hgrp):
            store_copy(per_w - 1, 1, g1).wait()

    return k(xt2d, pos_bc, table72)


def kernel(x, table):
    b_total, length = x.shape
    hidden = table.shape[1]
    pos = _pos_encoding(length, hidden)
    pos_bc = jnp.broadcast_to(pos[:, :, None], (length, hidden, _LANES))
    xt2d = x.T.reshape(length * b_total // _BLK, _BLK)
    table72 = jnp.pad(table, ((0, 0), (0, _PITCH - hidden)))
    out5 = _emb_lookup(
        xt2d, pos_bc, table72,
        b_total=b_total, hidden=hidden, length=length,
    )
    return (out5.transpose(2, 4, 0, 1, 3)
            .reshape(b_total, length, hidden))
